# baseline (speedup 1.0000x reference)
import jax
import jax.numpy as jnp
from jax import lax
from jax.experimental import pallas as pl
from jax.experimental.pallas import tpu as pltpu
from jax.experimental.pallas import tpu_sc as plsc

_CW = 2048
_BANDS = 8
_WPB = 4
_SLOTS = 4


def _sc_body(x_hbm, o_hbm, bufs, tailbuf, sem_in, sem_out, sem_tail):
    n = x_hbm.shape[1]
    fb = n // _CW
    tail = n - fb * _CW
    pw = fb // _WPB

    wid = lax.axis_index("s") * 2 + lax.axis_index("c")
    band = wid // _WPB
    r0 = 8 * band
    j0 = (wid % _WPB) * pw

    def in_copy(t, s):
        return pltpu.make_async_copy(
            x_hbm.at[pl.ds(r0, 8), pl.ds((j0 + t) * _CW, _CW)],
            bufs.at[s], sem_in.at[s])

    def out_copy(t, s):
        return pltpu.make_async_copy(
            bufs.at[s], o_hbm.at[pl.ds(r0, 8), pl.ds((j0 + t) * _CW, _CW)],
            sem_out.at[s])

    tr0 = 8 * wid

    def tail_in():
        return pltpu.make_async_copy(
            x_hbm.at[pl.ds(tr0, 8), pl.ds(fb * _CW, tail)], tailbuf, sem_tail)

    def tail_out():
        return pltpu.make_async_copy(
            tailbuf, o_hbm.at[pl.ds(tr0, 8), pl.ds(fb * _CW, tail)], sem_tail)

    @pl.when(wid < _BANDS)
    def _start_tail():
        tail_in().start()

    for j in range(min(2, pw)):
        in_copy(j, j).start()

    def step(t, _):
        j = t + 2

        @pl.when(j < pw)
        def _refill():
            s_j = lax.rem(j, _SLOTS)

            @pl.when(j >= _SLOTS)
            def _free():
                out_copy(j - _SLOTS, s_j).wait()

            in_copy(j, s_j).start()

        s = lax.rem(t, _SLOTS)
        in_copy(t, s).wait()

        @pl.when((t == 0) & (wid == 0))
        def _patch():
            v = bufs[0, 0, 0:16]
            lane = lax.broadcasted_iota(jnp.int32, (16,), 0)
            v = jnp.where(lane == 0, jnp.float32(88.0), v)
            v = jnp.where(lane == 1, jnp.float32(99.0), v)
            bufs[0, 0, 0:16] = v

        out_copy(t, s).start()
        return 0

    lax.fori_loop(0, pw, step, 0)

    for t in range(max(0, pw - _SLOTS), pw):
        out_copy(t, t % _SLOTS).wait()

    @pl.when(wid < _BANDS)
    def _finish_tail():
        tail_in().wait()
        tail_out().start()
        tail_out().wait()


def kernel(x):
    n, d = x.shape
    xt = x.T
    fb = n // _CW
    tail = n - fb * _CW
    mesh = plsc.VectorSubcoreMesh(core_axis_name="c", subcore_axis_name="s")
    f = pl.kernel(
        _sc_body,
        out_type=jax.ShapeDtypeStruct((d, n), x.dtype),
        mesh=mesh,
        scratch_types=[
            pltpu.VMEM((_SLOTS, 8, _CW), x.dtype),
            pltpu.VMEM((8, tail), x.dtype),
            pltpu.SemaphoreType.DMA((_SLOTS,)),
            pltpu.SemaphoreType.DMA((_SLOTS,)),
            pltpu.SemaphoreType.DMA,
        ],
    )
    return f(xt).T


# speedup vs baseline: 1.2541x; 1.2541x over previous
import jax
import jax.numpy as jnp
from jax.experimental import pallas as pl

_BLOCK_COLS = 49152


def _copy_scatter_body(x_ref, o_ref):
    o_ref[...] = x_ref[...]

    @pl.when(pl.program_id(0) == 0)
    def _patch_tile():
        sub = x_ref[0:8, 0:128]
        r = jax.lax.broadcasted_iota(jnp.int32, sub.shape, 0)
        c = jax.lax.broadcasted_iota(jnp.int32, sub.shape, 1)
        row0 = r == 0
        sub = jnp.where(row0 & (c == 0), jnp.float32(88.0), sub)
        sub = jnp.where(row0 & (c == 1), jnp.float32(99.0), sub)
        o_ref[0:8, 0:128] = sub


def kernel(x):
    n, d = x.shape
    xt = x.T
    grid = pl.cdiv(n, _BLOCK_COLS)
    out_t = pl.pallas_call(
        _copy_scatter_body,
        grid=(grid,),
        in_specs=[pl.BlockSpec((d, _BLOCK_COLS), lambda i: (0, i))],
        out_specs=pl.BlockSpec((d, _BLOCK_COLS), lambda i: (0, i)),
        out_shape=jax.ShapeDtypeStruct((d, n), x.dtype),
    )(xt)
    return out_t.T
